# baseline (speedup 1.0000x reference)
import functools
import math

import jax
import jax.numpy as jnp
from jax import lax
from jax.experimental import pallas as pl
from jax.experimental.pallas import tpu as pltpu
from jax.experimental.pallas import tpu_sc as plsc

_EMBED = 64
_LANES = 16
_VPR = _EMBED // _LANES
_SCALE = math.sqrt(_EMBED)

_info = plsc.get_sparse_core_info()
_NC, _NS = _info.num_cores, _info.num_subcores
_NW = _NC * _NS
_NBUF = 2


def _make_lookup(src_len: int, batch: int, vocab: int):
    cols_per_w = batch // _NW
    pair_w = 2 * _EMBED
    mesh = plsc.VectorSubcoreMesh(core_axis_name="c", subcore_axis_name="s")

    @functools.partial(
        pl.kernel,
        out_type=jax.ShapeDtypeStruct((src_len, batch, _EMBED), jnp.float32),
        mesh=mesh,
        scratch_types=[
            pltpu.VMEM((src_len, cols_per_w), jnp.int32),
            [pltpu.VMEM((cols_per_w,), jnp.int32) for _ in range(_NBUF)],
            [pltpu.VMEM((cols_per_w, pair_w), jnp.float32) for _ in range(_NBUF)],
            [pltpu.VMEM((cols_per_w, _EMBED), jnp.float32) for _ in range(_NBUF)],
            [pltpu.SemaphoreType.DMA for _ in range(_NBUF)],
            [pltpu.SemaphoreType.DMA for _ in range(_NBUF)],
        ],
    )
    def lookup(tok_hbm, pairs_hbm, out_hbm, idx_v, q_bufs, pair_bufs, stage_bufs, gsems, ssems):
        w = lax.axis_index("s") * _NC + lax.axis_index("c")
        col0 = w * cols_per_w
        pltpu.sync_copy(tok_hbm.at[:, pl.ds(col0, cols_per_w)], idx_v)

        def compute_q(s, b):
            for v in range(cols_per_w // _LANES):
                sl = pl.ds(v * _LANES, _LANES)
                q_bufs[b][sl] = lax.shift_right_logical(idx_v[s, sl], 1)

        def gather_desc(b):
            return pltpu.make_async_copy(
                pairs_hbm.at[q_bufs[b]], pair_bufs[b], gsems[b]
            )

        def out_desc(s, b):
            return pltpu.make_async_copy(
                stage_bufs[b], out_hbm.at[s, pl.ds(col0, cols_per_w)], ssems[b]
            )

        def scale(s, b):
            @pl.loop(0, cols_per_w // _LANES)
            def _group(g):
                base = g * _LANES
                t_vec = idx_v[s, pl.ds(base, _LANES)]
                p_vec = (t_vec & 1) * _EMBED
                for l in range(_LANES):
                    r = base + l
                    p = p_vec[l]
                    for j in range(_VPR):
                        stage_bufs[b][r, pl.ds(j * _LANES, _LANES)] = (
                            pair_bufs[b][r, pl.ds(p + j * _LANES, _LANES)] * _SCALE
                        )

        for b in range(_NBUF):
            compute_q(b, b)
            gather_desc(b).start()

        @pl.loop(0, src_len, step=_NBUF)
        def _pipeline(c0):
            for b in range(_NBUF):
                s = c0 + b
                gather_desc(b).wait()

                @pl.when(c0 >= _NBUF)
                def _():
                    out_desc(s - _NBUF, b).wait()

                scale(s, b)
                out_desc(s, b).start()

                @pl.when(c0 < src_len - _NBUF)
                def _():
                    compute_q(s + _NBUF, b)
                    gather_desc(b).start()

        for b in range(_NBUF):
            out_desc(src_len - _NBUF + b, b).wait()

    return lookup


def kernel(tokens, table):
    vocab, embed = table.shape
    src_len, batch = tokens.shape
    pairs = table.reshape(vocab // 2, 2 * embed)
    return _make_lookup(src_len, batch, vocab)(tokens.astype(jnp.int32), pairs)


# speedup vs baseline: 1.1504x; 1.1504x over previous
import functools
import math

import jax
import jax.numpy as jnp
from jax import lax
from jax.experimental import pallas as pl
from jax.experimental.pallas import tpu as pltpu
from jax.experimental.pallas import tpu_sc as plsc

_EMBED = 64
_LANES = 16
_VPR = _EMBED // _LANES
_SCALE = math.sqrt(_EMBED)

_info = plsc.get_sparse_core_info()
_NC, _NS = _info.num_cores, _info.num_subcores
_NW = _NC * _NS
_NBUF = 4


def _make_lookup(src_len: int, batch: int, vocab: int):
    cols_per_w = batch // _NW
    mesh = plsc.VectorSubcoreMesh(core_axis_name="c", subcore_axis_name="s")

    @functools.partial(
        pl.kernel,
        out_type=jax.ShapeDtypeStruct((src_len, batch, _EMBED), jnp.float32),
        mesh=mesh,
        scratch_types=[
            pltpu.VMEM((src_len, cols_per_w), jnp.int32),
            [pltpu.VMEM((cols_per_w, _EMBED), jnp.float32) for _ in range(_NBUF)],
            [pltpu.VMEM((cols_per_w, _EMBED), jnp.float32) for _ in range(_NBUF)],
            [pltpu.SemaphoreType.DMA for _ in range(_NBUF)],
            [pltpu.SemaphoreType.DMA for _ in range(_NBUF)],
        ],
        compiler_params=pltpu.CompilerParams(use_tc_tiling_on_sc=False),
    )
    def lookup(tok_hbm, table_hbm, out_hbm, idx_v, row_bufs, stage_bufs, gsems, ssems):
        w = lax.axis_index("s") * _NC + lax.axis_index("c")
        col0 = w * cols_per_w
        pltpu.sync_copy(tok_hbm.at[:, pl.ds(col0, cols_per_w)], idx_v)

        def gather_desc(s, b):
            return pltpu.make_async_copy(
                table_hbm.at[idx_v.at[s]], row_bufs[b], gsems[b]
            )

        def out_desc(s, b):
            return pltpu.make_async_copy(
                stage_bufs[b], out_hbm.at[s, pl.ds(col0, cols_per_w)], ssems[b]
            )

        def scale(b):
            @pl.loop(0, cols_per_w)
            def _row(r):
                for j in range(_VPR):
                    sl = pl.ds(j * _LANES, _LANES)
                    stage_bufs[b][r, sl] = row_bufs[b][r, sl] * _SCALE

        for b in range(_NBUF):
            gather_desc(b, b).start()

        @pl.loop(0, src_len, step=_NBUF)
        def _pipeline(c0):
            for b in range(_NBUF):
                s = c0 + b
                gather_desc(s, b).wait()

                @pl.when(c0 >= _NBUF)
                def _():
                    out_desc(s - _NBUF, b).wait()

                scale(b)
                out_desc(s, b).start()

                @pl.when(c0 < src_len - _NBUF)
                def _():
                    gather_desc(s + _NBUF, b).start()

        for b in range(_NBUF):
            out_desc(src_len - _NBUF + b, b).wait()

    return lookup


def kernel(tokens, table):
    src_len, batch = tokens.shape
    vocab, _ = table.shape
    return _make_lookup(src_len, batch, vocab)(tokens.astype(jnp.int32), table)


# speedup vs baseline: 1.2390x; 1.0770x over previous
import functools
import math

import jax
import jax.numpy as jnp
from jax import lax
from jax.experimental import pallas as pl
from jax.experimental.pallas import tpu as pltpu
from jax.experimental.pallas import tpu_sc as plsc

_EMBED = 64
_LANES = 16
_VPR = _EMBED // _LANES
_SCALE = math.sqrt(_EMBED)
_PAIR_W = 2 * _EMBED

_info = plsc.get_sparse_core_info()
_NC, _NS = _info.num_cores, _info.num_subcores
_NW = _NC * _NS
_NBUF = 2


def _make_lookup(src_len: int, batch: int, vocab: int):
    cols_per_w = batch // _NW
    mesh = plsc.VectorSubcoreMesh(core_axis_name="c", subcore_axis_name="s")

    @functools.partial(
        pl.kernel,
        out_type=jax.ShapeDtypeStruct((src_len, batch, _EMBED), jnp.float32),
        mesh=mesh,
        scratch_types=[
            pltpu.VMEM((src_len, cols_per_w), jnp.int32),
            [pltpu.VMEM((cols_per_w, _PAIR_W), jnp.float32) for _ in range(_NBUF)],
            [pltpu.VMEM((cols_per_w, _EMBED), jnp.float32) for _ in range(_NBUF)],
            [pltpu.SemaphoreType.DMA for _ in range(_NBUF)],
            [pltpu.SemaphoreType.DMA for _ in range(_NBUF)],
        ],
    )
    def lookup(tok_hbm, wide_hbm, out_hbm, idx_v, row_bufs, stage_bufs, gsems, ssems):
        w = lax.axis_index("s") * _NC + lax.axis_index("c")
        col0 = w * cols_per_w
        pltpu.sync_copy(tok_hbm.at[:, pl.ds(col0, cols_per_w)], idx_v)

        def gather_desc(s, b):
            return pltpu.make_async_copy(
                wide_hbm.at[idx_v.at[s]], row_bufs[b], gsems[b]
            )

        def out_desc(s, b):
            return pltpu.make_async_copy(
                stage_bufs[b], out_hbm.at[s, pl.ds(col0, cols_per_w)], ssems[b]
            )

        def scale(b):
            @pl.loop(0, cols_per_w)
            def _row(r):
                for j in range(_VPR):
                    sl = pl.ds(j * _LANES, _LANES)
                    stage_bufs[b][r, sl] = row_bufs[b][r, sl] * _SCALE

        for b in range(_NBUF):
            gather_desc(b, b).start()

        @pl.loop(0, src_len, step=_NBUF)
        def _pipeline(c0):
            for b in range(_NBUF):
                s = c0 + b
                gather_desc(s, b).wait()

                @pl.when(c0 >= _NBUF)
                def _():
                    out_desc(s - _NBUF, b).wait()

                scale(b)
                out_desc(s, b).start()

                @pl.when(c0 < src_len - _NBUF)
                def _():
                    gather_desc(s + _NBUF, b).start()

        for b in range(_NBUF):
            out_desc(src_len - _NBUF + b, b).wait()

    return lookup


def kernel(tokens, table):
    src_len, batch = tokens.shape
    vocab, embed = table.shape
    wide = jnp.pad(table, ((0, 0), (0, _PAIR_W - embed)))
    return _make_lookup(src_len, batch, vocab)(tokens.astype(jnp.int32), wide)
